# baseline (speedup 1.0000x reference)
import functools

import jax
import jax.numpy as jnp
from jax import lax
from jax.experimental import pallas as pl
from jax.experimental.pallas import tpu as pltpu
from jax.experimental.pallas import tpu_sc as plsc

N_NODES = 10000
H = 128
N_EDGES = 320000

NC = 2
NS = 16
NW = NC * NS
CHUNK = 50
NCHUNK = 200
NRES = 40
NBUF = 4
ROWS_PAD = 10112
ROWS_PER_TILE = ROWS_PAD // NS

W_EDGE = 1.0 / 32.0

ROW_BLK = 2000


def _k1_body(x_ref, w1_ref, b1_ref, w2_ref, b2_ref, o_ref):
    x = x_ref[...]
    h = jnp.dot(x, w1_ref[...], preferred_element_type=jnp.float32) + b1_ref[...]
    o_ref[...] = (
        jnp.dot(h, w2_ref[...], preferred_element_type=jnp.float32) + b2_ref[...]
    ) * W_EDGE


def _dense_in(n_feat, W_in, b_in, W_n2e, b_n2e):
    return pl.pallas_call(
        _k1_body,
        grid=(N_NODES // ROW_BLK,),
        in_specs=[
            pl.BlockSpec((ROW_BLK, H), lambda i: (i, 0)),
            pl.BlockSpec((H, H), lambda i: (0, 0)),
            pl.BlockSpec((1, H), lambda i: (0, 0)),
            pl.BlockSpec((H, H), lambda i: (0, 0)),
            pl.BlockSpec((1, H), lambda i: (0, 0)),
        ],
        out_specs=pl.BlockSpec((ROW_BLK, H), lambda i: (i, 0)),
        out_shape=jax.ShapeDtypeStruct((N_NODES, H), jnp.float32),
    )(n_feat, W_in, b_in.reshape(1, H), W_n2e, b_n2e.reshape(1, H))


def _k2_body(p_ref, w_ref, b_ref, a_ref, ef_ref, wh_ref):
    p = p_ref[0] + p_ref[1]
    a = a_ref[0]
    ef = jnp.where(p >= 0, p, a * p)
    ef_ref[...] = ef
    wh_ref[...] = (
        jnp.dot(ef, w_ref[...], preferred_element_type=jnp.float32) + b_ref[...]
    ) * W_EDGE


def _combine_prelu_matmul(partials, W_e2n, b_e2n, prelu_a):
    return pl.pallas_call(
        _k2_body,
        grid=(N_NODES // ROW_BLK,),
        in_specs=[
            pl.BlockSpec((2, ROW_BLK, H), lambda i: (0, i, 0)),
            pl.BlockSpec((H, H), lambda i: (0, 0)),
            pl.BlockSpec((1, H), lambda i: (0, 0)),
            pl.BlockSpec(memory_space=pltpu.SMEM),
        ],
        out_specs=[
            pl.BlockSpec((ROW_BLK, H), lambda i: (i, 0)),
            pl.BlockSpec((ROW_BLK, H), lambda i: (i, 0)),
        ],
        out_shape=[
            jax.ShapeDtypeStruct((N_NODES, H), jnp.float32),
            jax.ShapeDtypeStruct((N_NODES, H), jnp.float32),
        ],
    )(partials, W_e2n, b_e2n.reshape(1, H), prelu_a.reshape(1))


def _k3_body(p_ref, a_ref, o_ref):
    p = p_ref[0] + p_ref[1]
    a = a_ref[0]
    o_ref[...] = jnp.where(p >= 0, p, a * p)


def _combine_prelu(partials, prelu_a):
    return pl.pallas_call(
        _k3_body,
        grid=(N_NODES // ROW_BLK,),
        in_specs=[
            pl.BlockSpec((2, ROW_BLK, H), lambda i: (0, i, 0)),
            pl.BlockSpec(memory_space=pltpu.SMEM),
        ],
        out_specs=pl.BlockSpec((ROW_BLK, H), lambda i: (i, 0)),
        out_shape=jax.ShapeDtypeStruct((N_NODES, H), jnp.float32),
    )(partials, prelu_a.reshape(1))


def _sc_scatter(table, gat_idx, sct_idx, zeros_blk):
    mesh = plsc.VectorSubcoreMesh(core_axis_name="c", subcore_axis_name="s",
                                  num_cores=NC, num_subcores=NS)

    @functools.partial(
        pl.kernel,
        mesh=mesh,
        out_type=jax.ShapeDtypeStruct((NC * ROWS_PAD, H), jnp.float32),
        scratch_types=[
            pltpu.VMEM((NRES, CHUNK), jnp.int32),
            pltpu.VMEM((NRES, CHUNK), jnp.int32),
            pltpu.VMEM((NBUF, CHUNK, H), jnp.float32),
            pltpu.VMEM_SHARED((ROWS_PAD, H), jnp.float32),
            pltpu.SemaphoreType.DMA((NBUF,)),
            pltpu.SemaphoreType.DMA((NBUF,)),
        ],
    )
    def scatter_k(table_hbm, gat_hbm, sct_hbm, zeros_hbm, out_hbm,
                  src_v, dst_v, rows, acc, gsem, ssem):
        c = lax.axis_index("c")
        s = lax.axis_index("s")
        wid = c * NS + s
        row0 = s * ROWS_PER_TILE

        pltpu.sync_copy(zeros_hbm, acc.at[pl.ds(row0, ROWS_PER_TILE)])

        def gather(j, b):
            return pltpu.async_copy(table_hbm.at[src_v.at[j]], rows.at[b],
                                    gsem.at[b])

        def gather_wait(j, b):
            pltpu.make_async_copy(table_hbm.at[src_v.at[j]], rows.at[b],
                                  gsem.at[b]).wait()

        def scat(j, b):
            return pltpu.async_copy(rows.at[b], acc.at[dst_v.at[j]],
                                    ssem.at[b], add=True)

        def scat_wait(j, b):
            pltpu.make_async_copy(rows.at[b], acc.at[dst_v.at[j]],
                                  ssem.at[b]).wait()

        for half in range(NCHUNK // NRES):
            base = wid * NCHUNK + half * NRES
            pltpu.sync_copy(gat_hbm.at[pl.ds(base, NRES)], src_v)
            pltpu.sync_copy(sct_hbm.at[pl.ds(base, NRES)], dst_v)
            if half == 0:
                plsc.subcore_barrier()

            for b in range(NBUF):
                gather(b, b)

            def body(i, _):
                j = i * NBUF
                for b in range(NBUF):
                    gather_wait(j + b, b)
                    scat(j + b, b)
                for b in range(NBUF):
                    scat_wait(j + b, b)
                    gather(j + NBUF + b, b)
                return _

            lax.fori_loop(0, NRES // NBUF - 1, body, None)

            j = NRES - NBUF
            for b in range(NBUF):
                gather_wait(j + b, b)
                scat(j + b, b)
            for b in range(NBUF):
                scat_wait(j + b, b)
        plsc.subcore_barrier()

        pltpu.sync_copy(acc.at[pl.ds(row0, ROWS_PER_TILE)],
                        out_hbm.at[pl.ds(c * ROWS_PAD + row0, ROWS_PER_TILE)])

    return scatter_k(table, gat_idx, sct_idx, zeros_blk).reshape(NC, ROWS_PAD, H)


def kernel(n_feat, he_feat, edge_index, W_in, b_in, W_n2e, b_n2e, W_e2n, b_e2n, prelu_a):
    del he_feat

    src = edge_index[0].reshape(NW * NCHUNK, CHUNK)
    dst = edge_index[1].reshape(NW * NCHUNK, CHUNK)
    zeros_blk = jnp.zeros((ROWS_PER_TILE, H), jnp.float32)

    wh1 = _dense_in(n_feat, W_in, b_in, W_n2e, b_n2e)
    part1 = _sc_scatter(wh1, src, dst, zeros_blk)
    efeat, wh2 = _combine_prelu_matmul(part1, W_e2n, b_e2n, prelu_a)
    part2 = _sc_scatter(wh2, dst, src, zeros_blk)
    nfeat_out = _combine_prelu(part2, prelu_a)
    return (nfeat_out, efeat)


# speedup vs baseline: 1.0419x; 1.0419x over previous
import functools

import jax
import jax.numpy as jnp
from jax import lax
from jax.experimental import pallas as pl
from jax.experimental.pallas import tpu as pltpu
from jax.experimental.pallas import tpu_sc as plsc

N_NODES = 10000
H = 128
N_EDGES = 320000

NC = 2
NS = 16
NW = NC * NS
CHUNK = 50
NCHUNK = 200
NRES = 40
NBUF = 4
ROWS_PAD = 10112
ROWS_PER_TILE = ROWS_PAD // NS

W_EDGE = 1.0 / 32.0

ROW_BLK = 2000


def _k1_body(x_ref, w1_ref, b1_ref, w2_ref, b2_ref, o_ref):
    x = x_ref[...]
    h = jnp.dot(x, w1_ref[...], preferred_element_type=jnp.float32) + b1_ref[...]
    o_ref[...] = (
        jnp.dot(h, w2_ref[...], preferred_element_type=jnp.float32) + b2_ref[...]
    ) * W_EDGE


def _dense_in(n_feat, W_in, b_in, W_n2e, b_n2e):
    return pl.pallas_call(
        _k1_body,
        grid=(N_NODES // ROW_BLK,),
        in_specs=[
            pl.BlockSpec((ROW_BLK, H), lambda i: (i, 0)),
            pl.BlockSpec((H, H), lambda i: (0, 0)),
            pl.BlockSpec((1, H), lambda i: (0, 0)),
            pl.BlockSpec((H, H), lambda i: (0, 0)),
            pl.BlockSpec((1, H), lambda i: (0, 0)),
        ],
        out_specs=pl.BlockSpec((ROW_BLK, H), lambda i: (i, 0)),
        out_shape=jax.ShapeDtypeStruct((N_NODES, H), jnp.float32),
    )(n_feat, W_in, b_in.reshape(1, H), W_n2e, b_n2e.reshape(1, H))


def _k2_body(p_ref, w_ref, b_ref, a_ref, ef_ref, wh_ref):
    p = p_ref[0] + p_ref[1]
    a = a_ref[0]
    ef = jnp.where(p >= 0, p, a * p)
    ef_ref[...] = ef
    wh_ref[...] = (
        jnp.dot(ef, w_ref[...], preferred_element_type=jnp.float32) + b_ref[...]
    ) * W_EDGE


def _combine_prelu_matmul(partials, W_e2n, b_e2n, prelu_a):
    return pl.pallas_call(
        _k2_body,
        grid=(N_NODES // ROW_BLK,),
        in_specs=[
            pl.BlockSpec((2, ROW_BLK, H), lambda i: (0, i, 0)),
            pl.BlockSpec((H, H), lambda i: (0, 0)),
            pl.BlockSpec((1, H), lambda i: (0, 0)),
            pl.BlockSpec(memory_space=pltpu.SMEM),
        ],
        out_specs=[
            pl.BlockSpec((ROW_BLK, H), lambda i: (i, 0)),
            pl.BlockSpec((ROW_BLK, H), lambda i: (i, 0)),
        ],
        out_shape=[
            jax.ShapeDtypeStruct((N_NODES, H), jnp.float32),
            jax.ShapeDtypeStruct((N_NODES, H), jnp.float32),
        ],
    )(partials, W_e2n, b_e2n.reshape(1, H), prelu_a.reshape(1))


def _k3_body(p_ref, a_ref, o_ref):
    p = p_ref[0] + p_ref[1]
    a = a_ref[0]
    o_ref[...] = jnp.where(p >= 0, p, a * p)


def _combine_prelu(partials, prelu_a):
    return pl.pallas_call(
        _k3_body,
        grid=(N_NODES // ROW_BLK,),
        in_specs=[
            pl.BlockSpec((2, ROW_BLK, H), lambda i: (0, i, 0)),
            pl.BlockSpec(memory_space=pltpu.SMEM),
        ],
        out_specs=pl.BlockSpec((ROW_BLK, H), lambda i: (i, 0)),
        out_shape=jax.ShapeDtypeStruct((N_NODES, H), jnp.float32),
    )(partials, prelu_a.reshape(1))


def _sc_scatter(table, edges, gat_row, zeros_blk):
    sct_row = 1 - gat_row
    mesh = plsc.VectorSubcoreMesh(core_axis_name="c", subcore_axis_name="s",
                                  num_cores=NC, num_subcores=NS)

    @functools.partial(
        pl.kernel,
        mesh=mesh,
        out_type=jax.ShapeDtypeStruct((NC * ROWS_PAD, H), jnp.float32),
        scratch_types=[
            pltpu.VMEM((NRES, CHUNK), jnp.int32),
            pltpu.VMEM((NRES, CHUNK), jnp.int32),
            pltpu.VMEM((NBUF, CHUNK, H), jnp.float32),
            pltpu.VMEM_SHARED((ROWS_PAD, H), jnp.float32),
            pltpu.SemaphoreType.DMA((NBUF,)),
            pltpu.SemaphoreType.DMA((NBUF,)),
        ],
    )
    def scatter_k(table_hbm, edges_hbm, zeros_hbm, out_hbm,
                  src_v, dst_v, rows, acc, gsem, ssem):
        c = lax.axis_index("c")
        s = lax.axis_index("s")
        wid = c * NS + s
        row0 = s * ROWS_PER_TILE

        pltpu.sync_copy(zeros_hbm, acc.at[pl.ds(row0, ROWS_PER_TILE)])

        def gather(j, b):
            return pltpu.async_copy(table_hbm.at[src_v.at[j]], rows.at[b],
                                    gsem.at[b])

        def gather_wait(j, b):
            pltpu.make_async_copy(table_hbm.at[src_v.at[j]], rows.at[b],
                                  gsem.at[b]).wait()

        def scat(j, b):
            return pltpu.async_copy(rows.at[b], acc.at[dst_v.at[j]],
                                    ssem.at[b], add=True)

        def scat_wait(j, b):
            pltpu.make_async_copy(rows.at[b], acc.at[dst_v.at[j]],
                                  ssem.at[b]).wait()

        for half in range(NCHUNK // NRES):
            base = wid * NCHUNK + half * NRES
            pltpu.sync_copy(edges_hbm.at[gat_row, pl.ds(base, NRES)], src_v)
            pltpu.sync_copy(edges_hbm.at[sct_row, pl.ds(base, NRES)], dst_v)
            if half == 0:
                plsc.subcore_barrier()

            for b in range(NBUF):
                gather(b, b)

            def body(i, _):
                j = i * NBUF
                for b in range(NBUF):
                    gather_wait(j + b, b)
                    scat(j + b, b)
                for b in range(NBUF):
                    scat_wait(j + b, b)
                    gather(j + NBUF + b, b)
                return _

            lax.fori_loop(0, NRES // NBUF - 1, body, None)

            j = NRES - NBUF
            for b in range(NBUF):
                gather_wait(j + b, b)
                scat(j + b, b)
            for b in range(NBUF):
                scat_wait(j + b, b)
        plsc.subcore_barrier()

        pltpu.sync_copy(acc.at[pl.ds(row0, ROWS_PER_TILE)],
                        out_hbm.at[pl.ds(c * ROWS_PAD + row0, ROWS_PER_TILE)])

    return scatter_k(table, edges, zeros_blk).reshape(NC, ROWS_PAD, H)


def kernel(n_feat, he_feat, edge_index, W_in, b_in, W_n2e, b_n2e, W_e2n, b_e2n, prelu_a):
    del he_feat

    edges = edge_index.reshape(2, NW * NCHUNK, CHUNK)
    zeros_blk = jnp.zeros((ROWS_PER_TILE, H), jnp.float32)

    wh1 = _dense_in(n_feat, W_in, b_in, W_n2e, b_n2e)
    part1 = _sc_scatter(wh1, edges, 0, zeros_blk)
    efeat, wh2 = _combine_prelu_matmul(part1, W_e2n, b_e2n, prelu_a)
    part2 = _sc_scatter(wh2, edges, 1, zeros_blk)
    nfeat_out = _combine_prelu(part2, prelu_a)
    return (nfeat_out, efeat)
